# baseline (speedup 1.0000x reference)
import jax
import jax.numpy as jnp
from jax import lax
from jax.experimental import pallas as pl
from jax.experimental.pallas import tpu as pltpu
from jax.experimental.pallas import tpu_sc as plsc

NC, NS = 2, 16

N_LG = 160000
D = 128
N_NODES = 10000
NODE_IDX_ROWS = N_LG // 128
NODE_ROWS_PT = N_NODES // NS

E_LG = 640000
DE = 16
N_ESEG = 160000
ECHUNK = 1024
NECH = ECHUNK // 128
EDGE_IDX_ROWS = E_LG // 128
EDGE_CHUNKS = E_LG // ECHUNK
HALF = N_ESEG // 2
DUMP = 4096
EACC_ROWS = HALF + DUMP
EACC_ROWS_PT = EACC_ROWS // NS
EOUT_ROWS_PT = HALF // NS
ECNT_ROWS = N_ESEG + 64
ECNT_ROWS_PT = ECNT_ROWS // NS

_mesh = plsc.VectorSubcoreMesh(
    core_axis_name="c", subcore_axis_name="s", num_cores=NC, num_subcores=NS)

_sc_params = pltpu.CompilerParams(use_tc_tiling_on_sc=False)

_f32 = jnp.float32


def _fill(buf, rows, width, vec):
    def body(r, _):
        for k in range(width // 16):
            buf[r, pl.ds(k * 16, 16)] = vec
        return 0

    lax.fori_loop(0, rows, body, 0)


NODE_RPT = 624

_mesh1 = plsc.VectorSubcoreMesh(
    core_axis_name="c", subcore_axis_name="s", num_cores=1, num_subcores=NS)


def _make_node_body(col):
    def _node_body(x, idxf, sums, acc_s, buf0, buf1, iv0, iv1, sem0, sem1,
                   ssem):
        sid = lax.axis_index("s")
        bufs, ivs, sems = (buf0, buf1), (iv0, iv1), (sem0, sem1)

        zeros = jnp.zeros((16,), _f32)
        _fill(buf0, 128, D, zeros)
        zb = sid * NODE_RPT
        for t in range(4):
            pltpu.sync_copy(buf0, acc_s.at[pl.ds(zb + t * 128, 128)])

        @pl.when(sid == NS - 1)
        def _():
            pltpu.sync_copy(buf0, acc_s.at[pl.ds(zb + 512, 128)])

        @pl.when(sid != NS - 1)
        def _():
            pltpu.sync_copy(buf0.at[pl.ds(0, 112)],
                            acc_s.at[pl.ds(zb + 512, 112)])

        plsc.subcore_barrier()

        n_t = 78 + jnp.where(sid < 2, 1, 0)

        def issue(i, slot):
            j = sid + NS * i
            pltpu.async_copy(idxf.at[pl.ds(j * 128, 128)], ivs[slot],
                             sems[slot])
            pltpu.async_copy(x.at[pl.ds(j * 128, 128), pl.ds(col, D)],
                             bufs[slot], sems[slot])

        def wait_in(slot):
            pltpu.make_async_copy(idxf.at[pl.ds(0, 128)], ivs[slot],
                                  sems[slot]).wait()
            pltpu.make_async_copy(x.at[pl.ds(0, 128), pl.ds(0, D)],
                                  bufs[slot], sems[slot]).wait()

        issue(0, 0)

        def pair(io, _):
            for b in range(2):
                i = 2 * io + b

                @pl.when(i + 1 < n_t)
                def _():
                    issue(i + 1, 1 - b)

                @pl.when(i < n_t)
                def _():
                    wait_in(b)
                    pltpu.async_copy(bufs[b], acc_s.at[ivs[b]], ssem,
                                     add=True).wait()
            return 0

        lax.fori_loop(0, 40, pair, 0)
        plsc.subcore_barrier()

        ob = sid * NODE_RPT

        @pl.when(sid == NS - 1)
        def _():
            pltpu.sync_copy(acc_s.at[pl.ds(ob, 640)],
                            sums.at[pl.ds(ob, 640), :])

        @pl.when(sid != NS - 1)
        def _():
            pltpu.sync_copy(acc_s.at[pl.ds(ob, NODE_RPT)],
                            sums.at[pl.ds(ob, NODE_RPT), :])

    return _node_body


def _node_scatter_dir(x, idxf, col):
    return pl.kernel(
        _make_node_body(col),
        out_type=jax.ShapeDtypeStruct((N_NODES, D), _f32),
        mesh=_mesh1,
        scratch_types=[
            pltpu.VMEM_SHARED((N_NODES, D), _f32),
            pltpu.VMEM((128, D), _f32),
            pltpu.VMEM((128, D), _f32),
            pltpu.VMEM((128,), jnp.int32),
            pltpu.VMEM((128,), jnp.int32),
            pltpu.SemaphoreType.DMA,
            pltpu.SemaphoreType.DMA,
            pltpu.SemaphoreType.DMA,
        ],
    )(x, idxf)


@jax.jit
def _node_scatter(x, idx0f, idx1f):
    s0 = _node_scatter_dir(x, idx0f, D)
    s1 = _node_scatter_dir(x, idx1f, 0)
    return jnp.stack([s0, s1])


NCNT_ROWS = N_NODES + 16
NCNT_ROWS_PT = NCNT_ROWS // NS


def _cnt_body(eidx3, idx2, ones_c, zeros_c, ecnts, ncnts, acc_c, acc_n,
              ones8, ones16, zbuf, iv0, iv1, sem0, sem1, ssem):
    cid = lax.axis_index("c")
    sid = lax.axis_index("s")
    ivs, sems = (iv0, iv1), (sem0, sem1)

    pltpu.sync_copy(zeros_c, zbuf)
    pltpu.sync_copy(ones_c, ones8)
    zeros = jnp.zeros((16,), _f32)
    _fill(ones16, 128, 16, zeros)
    zb = sid * ECNT_ROWS_PT
    for t in range(ECNT_ROWS_PT // 512):
        pltpu.sync_copy(zbuf, acc_c.at[pl.ds(zb + t * 512, 512)])
    rem = ECNT_ROWS_PT % 512
    if rem:
        off = zb + (ECNT_ROWS_PT // 512) * 512
        pltpu.sync_copy(zbuf.at[pl.ds(0, rem)], acc_c.at[pl.ds(off, rem)])
    zn = sid * NCNT_ROWS_PT
    for t in range(NCNT_ROWS_PT // 128):
        pltpu.sync_copy(ones16, acc_n.at[pl.ds(zn + t * 128, 128)])
    rem = NCNT_ROWS_PT % 128
    if rem:
        off = zn + (NCNT_ROWS_PT // 128) * 128
        pltpu.sync_copy(ones16.at[pl.ds(0, rem)], acc_n.at[pl.ds(off, rem)])
    _fill(ones16, 128, 16, jnp.ones((16,), _f32))
    plsc.subcore_barrier()

    n_t = 39 + jnp.where(sid < 1, 1, 0)

    def issue(k, slot):
        c = sid + NS * k
        pltpu.async_copy(eidx3.at[cid, pl.ds(NECH * c, NECH), :], ivs[slot],
                         sems[slot])

    def wait_in(slot):
        pltpu.make_async_copy(eidx3.at[0, pl.ds(0, NECH), :], ivs[slot],
                              sems[slot]).wait()

    issue(0, 0)

    def pair(ko, _):
        for b in range(2):
            k = 2 * ko + b

            @pl.when(k + 1 < n_t)
            def _():
                issue(k + 1, 1 - b)

            @pl.when(k < n_t)
            def _():
                wait_in(b)
                ds = [pltpu.async_copy(ones8, acc_c.at[ivs[b].at[q]], ssem,
                                       add=True) for q in range(NECH)]
                for dd in ds:
                    dd.wait()
        return 0

    lax.fori_loop(0, 20, pair, 0)

    n_t2 = 9 + jnp.where(sid < 12, 1, 0)

    def issue2(i, slot):
        ch = sid + NS * i
        pltpu.async_copy(idx2.at[cid, pl.ds(8 * ch, 8), :], ivs[slot],
                         sems[slot])

    def wait_in2(slot):
        pltpu.make_async_copy(idx2.at[0, pl.ds(0, 8), :], ivs[slot],
                              sems[slot]).wait()

    issue2(0, 0)

    def pair2(io, _):
        for b in range(2):
            i = 2 * io + b

            @pl.when(i + 1 < n_t2)
            def _():
                issue2(i + 1, 1 - b)

            @pl.when(i < n_t2)
            def _():
                wait_in2(b)
                ds = [pltpu.async_copy(ones16, acc_n.at[ivs[b].at[q]], ssem,
                                       add=True) for q in range(8)]
                for dd in ds:
                    dd.wait()
        return 0

    lax.fori_loop(0, 5, pair2, 0)

    @pl.when(sid < 2)
    def _():
        pltpu.sync_copy(idx2.at[cid, pl.ds(1248 + sid, 1), :],
                        iv0.at[pl.ds(0, 1), :])
        pltpu.sync_copy(ones16, acc_n.at[iv0.at[0]], add=True)

    plsc.subcore_barrier()

    ob = sid * (N_ESEG // NS)
    pltpu.sync_copy(acc_c.at[pl.ds(ob, N_ESEG // NS)],
                    ecnts.at[cid, pl.ds(ob, N_ESEG // NS), :])
    on = sid * NODE_ROWS_PT
    pltpu.sync_copy(acc_n.at[pl.ds(on, NODE_ROWS_PT)],
                    ncnts.at[cid, pl.ds(on, NODE_ROWS_PT), :])


@jax.jit
def _counts(eidx3, idx2, ones_c, zeros_c):
    return pl.kernel(
        _cnt_body,
        out_type=(jax.ShapeDtypeStruct((NC, N_ESEG, 8), _f32),
                  jax.ShapeDtypeStruct((NC, N_NODES, 16), _f32)),
        mesh=_mesh,
        scratch_types=[
            pltpu.VMEM_SHARED((ECNT_ROWS, 8), _f32),
            pltpu.VMEM_SHARED((NCNT_ROWS, 16), _f32),
            pltpu.VMEM((128, 8), _f32),
            pltpu.VMEM((128, 16), _f32),
            pltpu.VMEM((512, 8), _f32),
            pltpu.VMEM((NECH, 128), jnp.int32),
            pltpu.VMEM((NECH, 128), jnp.int32),
            pltpu.SemaphoreType.DMA,
            pltpu.SemaphoreType.DMA,
            pltpu.SemaphoreType.DMA,
        ],
        compiler_params=_sc_params,
    )(eidx3, idx2, ones_c, zeros_c)


def _edge_body(ea, eidx3, sums, acc_s, buf0, buf1, iv0, iv1, sem0, sem1,
               ssem):
    cid = lax.axis_index("c")
    sid = lax.axis_index("s")
    bufs, ivs, sems = (buf0, buf1), (iv0, iv1), (sem0, sem1)

    zeros = jnp.zeros((16,), _f32)
    n_t = 39 + jnp.where(sid < 1, 1, 0)

    for p in range(2):
        _fill(buf0, 512, DE, zeros)
        zb = sid * EACC_ROWS_PT
        for t in range(EACC_ROWS_PT // 512):
            pltpu.sync_copy(buf0.at[pl.ds(0, 512)],
                            acc_s.at[pl.ds(zb + t * 512, 512)])
        rem = EACC_ROWS_PT % 512
        if rem:
            off = zb + (EACC_ROWS_PT // 512) * 512
            pltpu.sync_copy(buf0.at[pl.ds(0, rem)], acc_s.at[pl.ds(off, rem)])
        plsc.subcore_barrier()

        def issue(k, slot):
            c = sid + NS * k
            pltpu.async_copy(eidx3.at[cid, pl.ds(NECH * c, NECH), :],
                             ivs[slot], sems[slot])

            @pl.when(cid == 0)
            def _():
                pltpu.async_copy(ea.at[pl.ds(ECHUNK * c, ECHUNK),
                                       pl.ds(DE, DE)], bufs[slot], sems[slot])

            @pl.when(cid == 1)
            def _():
                pltpu.async_copy(ea.at[pl.ds(ECHUNK * c, ECHUNK),
                                       pl.ds(0, DE)], bufs[slot], sems[slot])

        def wait_in(slot):
            pltpu.make_async_copy(eidx3.at[0, pl.ds(0, NECH), :], ivs[slot],
                                  sems[slot]).wait()
            pltpu.make_async_copy(ea.at[pl.ds(0, ECHUNK), pl.ds(0, DE)],
                                  bufs[slot], sems[slot]).wait()

        def transform(slot):
            for q in range(NECH):
                for m in range(8):
                    v = ivs[slot][q, pl.ds(m * 16, 16)]
                    if p == 0:
                        ing = v < HALF
                        local = v
                    else:
                        ing = v >= HALF
                        local = v - HALF
                    dump = HALF + lax.bitwise_and(v, DUMP - 1)
                    ivs[slot][q, pl.ds(m * 16, 16)] = jnp.where(ing, local,
                                                                dump)

        issue(0, 0)

        def pair(ko, _):
            for b in range(2):
                k = 2 * ko + b

                @pl.when(k + 1 < n_t)
                def _():
                    issue(k + 1, 1 - b)

                @pl.when(k < n_t)
                def _():
                    wait_in(b)
                    transform(b)
                    ds = [pltpu.async_copy(bufs[b].at[pl.ds(q * 128, 128)],
                                           acc_s.at[ivs[b].at[q]], ssem,
                                           add=True) for q in range(NECH)]
                    for dd in ds:
                        dd.wait()
            return 0

        lax.fori_loop(0, 20, pair, 0)
        plsc.subcore_barrier()

        ob = sid * EOUT_ROWS_PT
        pltpu.sync_copy(acc_s.at[pl.ds(ob, EOUT_ROWS_PT)],
                        sums.at[cid, pl.ds(p * HALF + ob, EOUT_ROWS_PT), :])
        plsc.subcore_barrier()


@jax.jit
def _edge_scatter(ea, eidx3):
    return pl.kernel(
        _edge_body,
        out_type=jax.ShapeDtypeStruct((NC, N_ESEG, DE), _f32),
        mesh=_mesh,
        scratch_types=[
            pltpu.VMEM_SHARED((EACC_ROWS, DE), _f32),
            pltpu.VMEM((ECHUNK, DE), _f32),
            pltpu.VMEM((ECHUNK, DE), _f32),
            pltpu.VMEM((NECH, 128), jnp.int32),
            pltpu.VMEM((NECH, 128), jnp.int32),
            pltpu.SemaphoreType.DMA,
            pltpu.SemaphoreType.DMA,
            pltpu.SemaphoreType.DMA,
        ],
        compiler_params=_sc_params,
    )(ea, eidx3)


def _combine_body(s_ref, c_ref, o_ref):
    s0, s1 = s_ref[0], s_ref[1]
    c0 = c_ref[0][:, 0:1]
    c1 = c_ref[1][:, 0:1]
    o_ref[...] = (s0 / jnp.maximum(c0, 1.0)) + (s1 / jnp.maximum(c1, 1.0))


def _combine(sums, cnts, rows_blk, n_rows, width, cwidth):
    grid = n_rows // rows_blk
    return pl.pallas_call(
        _combine_body,
        grid=(grid,),
        in_specs=[
            pl.BlockSpec((NC, rows_blk, width), lambda i: (0, i, 0)),
            pl.BlockSpec((NC, rows_blk, cwidth), lambda i: (0, i, 0)),
        ],
        out_specs=pl.BlockSpec((rows_blk, width), lambda i: (i, 0)),
        out_shape=jax.ShapeDtypeStruct((n_rows, width), _f32),
    )(sums, cnts)


def kernel(x, lg_node_idx, edge_attr, edge_index, org_edge_attr, org_x,
           org_edge_index):
    idx0f = lg_node_idx[:, 0].astype(jnp.int32)
    idx1f = lg_node_idx[:, 1].astype(jnp.int32)
    idx2 = jnp.stack([idx0f.reshape(NODE_IDX_ROWS, 128),
                      idx1f.reshape(NODE_IDX_ROWS, 128)])
    eidx3 = edge_index.astype(jnp.int32).reshape(2, EDGE_IDX_ROWS, 128)
    ones_c = jnp.ones((128, 8), _f32)
    zeros_c = jnp.zeros((512, 8), _f32)

    nsums = _node_scatter(x, idx0f, idx1f)
    ecnts, ncnts = _counts(eidx3, idx2, ones_c, zeros_c)
    esums = _edge_scatter(edge_attr, eidx3)

    new_x = _combine(nsums, ncnts, 1000, N_NODES, D, 16)
    new_edge_attr = _combine(esums, ecnts, 1000, N_ESEG, DE, 8)
    return new_x, new_edge_attr, org_edge_index


# speedup vs baseline: 1.0045x; 1.0045x over previous
import jax
import jax.numpy as jnp
from jax import lax
from jax.experimental import pallas as pl
from jax.experimental.pallas import tpu as pltpu
from jax.experimental.pallas import tpu_sc as plsc

NC, NS = 2, 16

N_LG = 160000
D = 128
N_NODES = 10000
NODE_IDX_ROWS = N_LG // 128
NODE_ROWS_PT = N_NODES // NS

E_LG = 640000
DE = 16
N_ESEG = 160000
ECHUNK = 1024
NECH = ECHUNK // 128
EDGE_IDX_ROWS = E_LG // 128
EDGE_CHUNKS = E_LG // ECHUNK
HALF = N_ESEG // 2
DUMP = 4096
EACC_ROWS = HALF + DUMP
EACC_ROWS_PT = EACC_ROWS // NS
EOUT_ROWS_PT = HALF // NS
ECNT_ROWS = N_ESEG + 64
ECNT_ROWS_PT = ECNT_ROWS // NS

_mesh = plsc.VectorSubcoreMesh(
    core_axis_name="c", subcore_axis_name="s", num_cores=NC, num_subcores=NS)

_sc_params = pltpu.CompilerParams(use_tc_tiling_on_sc=False)

_f32 = jnp.float32


def _fill(buf, rows, width, vec):
    def body(r, _):
        for k in range(width // 16):
            buf[r, pl.ds(k * 16, 16)] = vec
        return 0

    lax.fori_loop(0, rows, body, 0)


NODE_RPT = 624


def _node_body(x, idx0f, idx1f, sums, acc_s, buf0, buf1, iv0, iv1, sem0,
               sem1, ssem):
    cid = lax.axis_index("c")
    sid = lax.axis_index("s")
    bufs, ivs, sems = (buf0, buf1), (iv0, iv1), (sem0, sem1)

    zeros = jnp.zeros((16,), _f32)
    _fill(buf0, 128, D, zeros)
    zb = sid * NODE_RPT
    for t in range(4):
        pltpu.sync_copy(buf0, acc_s.at[pl.ds(zb + t * 128, 128)])

    @pl.when(sid == NS - 1)
    def _():
        pltpu.sync_copy(buf0, acc_s.at[pl.ds(zb + 512, 128)])

    @pl.when(sid != NS - 1)
    def _():
        pltpu.sync_copy(buf0.at[pl.ds(0, 112)], acc_s.at[pl.ds(zb + 512, 112)])

    plsc.subcore_barrier()

    n_t = 78 + jnp.where(sid < 2, 1, 0)

    def issue(i, slot):
        j = sid + NS * i

        @pl.when(cid == 0)
        def _():
            pltpu.async_copy(idx0f.at[pl.ds(j * 128, 128)], ivs[slot],
                             sems[slot])
            pltpu.async_copy(x.at[pl.ds(j * 128, 128), pl.ds(D, D)],
                             bufs[slot], sems[slot])

        @pl.when(cid == 1)
        def _():
            pltpu.async_copy(idx1f.at[pl.ds(j * 128, 128)], ivs[slot],
                             sems[slot])
            pltpu.async_copy(x.at[pl.ds(j * 128, 128), pl.ds(0, D)],
                             bufs[slot], sems[slot])

    def wait_in(slot):
        pltpu.make_async_copy(idx0f.at[pl.ds(0, 128)], ivs[slot],
                              sems[slot]).wait()
        pltpu.make_async_copy(x.at[pl.ds(0, 128), pl.ds(0, D)], bufs[slot],
                              sems[slot]).wait()

    issue(0, 0)

    def pair(io, _):
        for b in range(2):
            i = 2 * io + b

            @pl.when(i + 1 < n_t)
            def _():
                issue(i + 1, 1 - b)

            @pl.when(i < n_t)
            def _():
                wait_in(b)
                pltpu.async_copy(bufs[b], acc_s.at[ivs[b]], ssem,
                                 add=True).wait()
        return 0

    lax.fori_loop(0, 40, pair, 0)
    plsc.subcore_barrier()

    ob = sid * NODE_RPT

    @pl.when(sid == NS - 1)
    def _():
        pltpu.sync_copy(acc_s.at[pl.ds(ob, 640)],
                        sums.at[cid, pl.ds(ob, 640), :])

    @pl.when(sid != NS - 1)
    def _():
        pltpu.sync_copy(acc_s.at[pl.ds(ob, NODE_RPT)],
                        sums.at[cid, pl.ds(ob, NODE_RPT), :])


@jax.jit
def _node_scatter(x, idx0f, idx1f):
    return pl.kernel(
        _node_body,
        out_type=jax.ShapeDtypeStruct((NC, N_NODES, D), _f32),
        mesh=_mesh,
        scratch_types=[
            pltpu.VMEM_SHARED((N_NODES, D), _f32),
            pltpu.VMEM((128, D), _f32),
            pltpu.VMEM((128, D), _f32),
            pltpu.VMEM((128,), jnp.int32),
            pltpu.VMEM((128,), jnp.int32),
            pltpu.SemaphoreType.DMA,
            pltpu.SemaphoreType.DMA,
            pltpu.SemaphoreType.DMA,
        ],
    )(x, idx0f, idx1f)


NCNT_ROWS = N_NODES + 16
NCNT_ROWS_PT = NCNT_ROWS // NS


def _cnt_body(eidx3, idx2, ones_c, zeros_c, ecnts, ncnts, acc_c, acc_n,
              ones8, ones16, zbuf, iv0, iv1, sem0, sem1, ssem):
    cid = lax.axis_index("c")
    sid = lax.axis_index("s")
    ivs, sems = (iv0, iv1), (sem0, sem1)

    pltpu.sync_copy(zeros_c, zbuf)
    pltpu.sync_copy(ones_c, ones8)
    zeros = jnp.zeros((16,), _f32)
    _fill(ones16, 128, 16, zeros)
    zb = sid * ECNT_ROWS_PT
    for t in range(ECNT_ROWS_PT // 512):
        pltpu.sync_copy(zbuf, acc_c.at[pl.ds(zb + t * 512, 512)])
    rem = ECNT_ROWS_PT % 512
    if rem:
        off = zb + (ECNT_ROWS_PT // 512) * 512
        pltpu.sync_copy(zbuf.at[pl.ds(0, rem)], acc_c.at[pl.ds(off, rem)])
    zn = sid * NCNT_ROWS_PT
    for t in range(NCNT_ROWS_PT // 128):
        pltpu.sync_copy(ones16, acc_n.at[pl.ds(zn + t * 128, 128)])
    rem = NCNT_ROWS_PT % 128
    if rem:
        off = zn + (NCNT_ROWS_PT // 128) * 128
        pltpu.sync_copy(ones16.at[pl.ds(0, rem)], acc_n.at[pl.ds(off, rem)])
    _fill(ones16, 128, 16, jnp.ones((16,), _f32))
    plsc.subcore_barrier()

    n_t = 39 + jnp.where(sid < 1, 1, 0)

    def issue(k, slot):
        c = sid + NS * k
        pltpu.async_copy(eidx3.at[cid, pl.ds(NECH * c, NECH), :], ivs[slot],
                         sems[slot])

    def wait_in(slot):
        pltpu.make_async_copy(eidx3.at[0, pl.ds(0, NECH), :], ivs[slot],
                              sems[slot]).wait()

    issue(0, 0)

    def pair(ko, _):
        for b in range(2):
            k = 2 * ko + b

            @pl.when(k + 1 < n_t)
            def _():
                issue(k + 1, 1 - b)

            @pl.when(k < n_t)
            def _():
                wait_in(b)
                ds = [pltpu.async_copy(ones8, acc_c.at[ivs[b].at[q]], ssem,
                                       add=True) for q in range(NECH)]
                for dd in ds:
                    dd.wait()
        return 0

    lax.fori_loop(0, 20, pair, 0)

    n_t2 = 9 + jnp.where(sid < 12, 1, 0)

    def issue2(i, slot):
        ch = sid + NS * i
        pltpu.async_copy(idx2.at[cid, pl.ds(8 * ch, 8), :], ivs[slot],
                         sems[slot])

    def wait_in2(slot):
        pltpu.make_async_copy(idx2.at[0, pl.ds(0, 8), :], ivs[slot],
                              sems[slot]).wait()

    issue2(0, 0)

    def pair2(io, _):
        for b in range(2):
            i = 2 * io + b

            @pl.when(i + 1 < n_t2)
            def _():
                issue2(i + 1, 1 - b)

            @pl.when(i < n_t2)
            def _():
                wait_in2(b)
                ds = [pltpu.async_copy(ones16, acc_n.at[ivs[b].at[q]], ssem,
                                       add=True) for q in range(8)]
                for dd in ds:
                    dd.wait()
        return 0

    lax.fori_loop(0, 5, pair2, 0)

    @pl.when(sid < 2)
    def _():
        pltpu.sync_copy(idx2.at[cid, pl.ds(1248 + sid, 1), :],
                        iv0.at[pl.ds(0, 1), :])
        pltpu.sync_copy(ones16, acc_n.at[iv0.at[0]], add=True)

    plsc.subcore_barrier()

    ob = sid * (N_ESEG // NS)
    pltpu.sync_copy(acc_c.at[pl.ds(ob, N_ESEG // NS)],
                    ecnts.at[cid, pl.ds(ob, N_ESEG // NS), :])
    on = sid * NODE_ROWS_PT
    pltpu.sync_copy(acc_n.at[pl.ds(on, NODE_ROWS_PT)],
                    ncnts.at[cid, pl.ds(on, NODE_ROWS_PT), :])


@jax.jit
def _counts(eidx3, idx2, ones_c, zeros_c):
    return pl.kernel(
        _cnt_body,
        out_type=(jax.ShapeDtypeStruct((NC, N_ESEG, 8), _f32),
                  jax.ShapeDtypeStruct((NC, N_NODES, 16), _f32)),
        mesh=_mesh,
        scratch_types=[
            pltpu.VMEM_SHARED((ECNT_ROWS, 8), _f32),
            pltpu.VMEM_SHARED((NCNT_ROWS, 16), _f32),
            pltpu.VMEM((128, 8), _f32),
            pltpu.VMEM((128, 16), _f32),
            pltpu.VMEM((512, 8), _f32),
            pltpu.VMEM((NECH, 128), jnp.int32),
            pltpu.VMEM((NECH, 128), jnp.int32),
            pltpu.SemaphoreType.DMA,
            pltpu.SemaphoreType.DMA,
            pltpu.SemaphoreType.DMA,
        ],
        compiler_params=_sc_params,
    )(eidx3, idx2, ones_c, zeros_c)


def _edge_body(ea, eidx3, sums, acc_s, buf0, buf1, iv0, iv1, sem0, sem1,
               ssem):
    cid = lax.axis_index("c")
    sid = lax.axis_index("s")
    bufs, ivs, sems = (buf0, buf1), (iv0, iv1), (sem0, sem1)

    zeros = jnp.zeros((16,), _f32)
    n_t = 39 + jnp.where(sid < 1, 1, 0)

    for p in range(2):
        _fill(buf0, 512, DE, zeros)
        zb = sid * EACC_ROWS_PT
        for t in range(EACC_ROWS_PT // 512):
            pltpu.sync_copy(buf0.at[pl.ds(0, 512)],
                            acc_s.at[pl.ds(zb + t * 512, 512)])
        rem = EACC_ROWS_PT % 512
        if rem:
            off = zb + (EACC_ROWS_PT // 512) * 512
            pltpu.sync_copy(buf0.at[pl.ds(0, rem)], acc_s.at[pl.ds(off, rem)])
        plsc.subcore_barrier()

        def issue(k, slot):
            c = sid + NS * k
            pltpu.async_copy(eidx3.at[cid, pl.ds(NECH * c, NECH), :],
                             ivs[slot], sems[slot])

            @pl.when(cid == 0)
            def _():
                pltpu.async_copy(ea.at[pl.ds(ECHUNK * c, ECHUNK),
                                       pl.ds(DE, DE)], bufs[slot], sems[slot])

            @pl.when(cid == 1)
            def _():
                pltpu.async_copy(ea.at[pl.ds(ECHUNK * c, ECHUNK),
                                       pl.ds(0, DE)], bufs[slot], sems[slot])

        def wait_in(slot):
            pltpu.make_async_copy(eidx3.at[0, pl.ds(0, NECH), :], ivs[slot],
                                  sems[slot]).wait()
            pltpu.make_async_copy(ea.at[pl.ds(0, ECHUNK), pl.ds(0, DE)],
                                  bufs[slot], sems[slot]).wait()

        def transform(slot):
            for q in range(NECH):
                for m in range(8):
                    v = ivs[slot][q, pl.ds(m * 16, 16)]
                    if p == 0:
                        ing = v < HALF
                        local = v
                    else:
                        ing = v >= HALF
                        local = v - HALF
                    dump = HALF + lax.bitwise_and(v, DUMP - 1)
                    ivs[slot][q, pl.ds(m * 16, 16)] = jnp.where(ing, local,
                                                                dump)

        issue(0, 0)

        def pair(ko, _):
            for b in range(2):
                k = 2 * ko + b

                @pl.when(k + 1 < n_t)
                def _():
                    issue(k + 1, 1 - b)

                @pl.when(k < n_t)
                def _():
                    wait_in(b)
                    transform(b)
                    ds = [pltpu.async_copy(bufs[b].at[pl.ds(q * 128, 128)],
                                           acc_s.at[ivs[b].at[q]], ssem,
                                           add=True) for q in range(NECH)]
                    for dd in ds:
                        dd.wait()
            return 0

        lax.fori_loop(0, 20, pair, 0)
        plsc.subcore_barrier()

        ob = sid * EOUT_ROWS_PT
        pltpu.sync_copy(acc_s.at[pl.ds(ob, EOUT_ROWS_PT)],
                        sums.at[cid, pl.ds(p * HALF + ob, EOUT_ROWS_PT), :])
        plsc.subcore_barrier()


@jax.jit
def _edge_scatter(ea, eidx3):
    return pl.kernel(
        _edge_body,
        out_type=jax.ShapeDtypeStruct((NC, N_ESEG, DE), _f32),
        mesh=_mesh,
        scratch_types=[
            pltpu.VMEM_SHARED((EACC_ROWS, DE), _f32),
            pltpu.VMEM((ECHUNK, DE), _f32),
            pltpu.VMEM((ECHUNK, DE), _f32),
            pltpu.VMEM((NECH, 128), jnp.int32),
            pltpu.VMEM((NECH, 128), jnp.int32),
            pltpu.SemaphoreType.DMA,
            pltpu.SemaphoreType.DMA,
            pltpu.SemaphoreType.DMA,
        ],
        compiler_params=_sc_params,
    )(ea, eidx3)


def _combine_body(s_ref, c_ref, o_ref):
    s0, s1 = s_ref[0], s_ref[1]
    c0 = c_ref[0][:, 0:1]
    c1 = c_ref[1][:, 0:1]
    o_ref[...] = (s0 / jnp.maximum(c0, 1.0)) + (s1 / jnp.maximum(c1, 1.0))


def _combine(sums, cnts, rows_blk, n_rows, width, cwidth):
    grid = n_rows // rows_blk
    return pl.pallas_call(
        _combine_body,
        grid=(grid,),
        in_specs=[
            pl.BlockSpec((NC, rows_blk, width), lambda i: (0, i, 0)),
            pl.BlockSpec((NC, rows_blk, cwidth), lambda i: (0, i, 0)),
        ],
        out_specs=pl.BlockSpec((rows_blk, width), lambda i: (i, 0)),
        out_shape=jax.ShapeDtypeStruct((n_rows, width), _f32),
    )(sums, cnts)


def kernel(x, lg_node_idx, edge_attr, edge_index, org_edge_attr, org_x,
           org_edge_index):
    idx0f = lg_node_idx[:, 0].astype(jnp.int32)
    idx1f = lg_node_idx[:, 1].astype(jnp.int32)
    idx2 = jnp.stack([idx0f.reshape(NODE_IDX_ROWS, 128),
                      idx1f.reshape(NODE_IDX_ROWS, 128)])
    eidx3 = edge_index.astype(jnp.int32).reshape(2, EDGE_IDX_ROWS, 128)
    ones_c = jnp.ones((128, 8), _f32)
    zeros_c = jnp.zeros((512, 8), _f32)

    nsums = _node_scatter(x, idx0f, idx1f)
    ecnts, ncnts = _counts(eidx3, idx2, ones_c, zeros_c)
    esums = _edge_scatter(edge_attr, eidx3)

    new_x = _combine(nsums, ncnts, 1000, N_NODES, D, 16)
    new_edge_attr = _combine(esums, ecnts, 1000, N_ESEG, DE, 8)
    return new_x, new_edge_attr, org_edge_index
